# baseline (speedup 1.0000x reference)
import functools

import jax
import jax.numpy as jnp
from jax import lax
from jax.experimental import pallas as pl
from jax.experimental.pallas import tpu as pltpu
from jax.experimental.pallas import tpu_sc as plsc

BATCH = 16384
DIM = 64
IDX_ROWS = 128
IDX_COLS = 128
ROWS_PER_W = 4

MLP_BLOCK = 2048


def _sc_gather_body(uidx_hbm, iidx_hbm, ue_hbm, ie_hbm, u_out, i_out,
                    idx_v, rows_v, sem):
    nc = 2
    wid = lax.axis_index("s") * nc + lax.axis_index("c")
    base = wid * ROWS_PER_W
    pltpu.sync_copy(uidx_hbm.at[pl.ds(base, ROWS_PER_W)], idx_v.at[0])
    pltpu.sync_copy(iidx_hbm.at[pl.ds(base, ROWS_PER_W)], idx_v.at[1])
    for t, (tab, out) in enumerate(((ue_hbm, u_out), (ie_hbm, i_out))):
        copies = [
            pltpu.async_copy(tab.at[idx_v.at[t].at[j]], rows_v.at[j], sem)
            for j in range(ROWS_PER_W)
        ]
        for c in copies:
            c.wait()
        pltpu.sync_copy(rows_v, out.at[pl.ds(base, ROWS_PER_W)])


def _sc_gather(uidx2d, iidx2d, ue2, ie2):
    mesh = plsc.VectorSubcoreMesh(core_axis_name="c", subcore_axis_name="s")
    kern = functools.partial(
        pl.kernel,
        mesh=mesh,
        out_type=[
            jax.ShapeDtypeStruct((IDX_ROWS, IDX_COLS, 2 * DIM), jnp.float32),
            jax.ShapeDtypeStruct((IDX_ROWS, IDX_COLS, 2 * DIM), jnp.float32),
        ],
        scratch_types=[
            pltpu.VMEM((2, ROWS_PER_W, IDX_COLS), jnp.int32),
            pltpu.VMEM((ROWS_PER_W, IDX_COLS, 2 * DIM), jnp.float32),
            pltpu.SemaphoreType.DMA,
        ],
    )(_sc_gather_body)
    return kern(uidx2d, iidx2d, ue2, ie2)


def _mlp_body(u_ref, i_ref, up_ref, ip_ref, w1a_ref, w1b_ref, b1_ref,
              w2_ref, b2_ref, w3_ref, b3_ref, o_ref):
    u = jnp.where(up_ref[...] == 1, u_ref[:, DIM:], u_ref[:, :DIM])
    i = jnp.where(ip_ref[...] == 1, i_ref[:, DIM:], i_ref[:, :DIM])
    h = jnp.dot(u, w1a_ref[...], preferred_element_type=jnp.float32)
    h = h + jnp.dot(i, w1b_ref[...], preferred_element_type=jnp.float32)
    h = jnp.maximum(h + b1_ref[...], 0.0)
    h = jnp.dot(h, w2_ref[...], preferred_element_type=jnp.float32) + b2_ref[...]
    h = jnp.maximum(h, 0.0)
    o = jnp.dot(h, w3_ref[...], preferred_element_type=jnp.float32) + b3_ref[...]
    o_ref[...] = 1.0 / (1.0 + jnp.exp(-o))


def _mlp(u_rows, i_rows, u_par, i_par, W1a, W1b, b1, W2, b2, W3, b3):
    n_blocks = BATCH // MLP_BLOCK
    return pl.pallas_call(
        _mlp_body,
        grid=(n_blocks,),
        in_specs=[
            pl.BlockSpec((MLP_BLOCK, 2 * DIM), lambda i: (i, 0)),
            pl.BlockSpec((MLP_BLOCK, 2 * DIM), lambda i: (i, 0)),
            pl.BlockSpec((MLP_BLOCK, 1), lambda i: (i, 0)),
            pl.BlockSpec((MLP_BLOCK, 1), lambda i: (i, 0)),
            pl.BlockSpec((DIM, 32), lambda i: (0, 0)),
            pl.BlockSpec((DIM, 32), lambda i: (0, 0)),
            pl.BlockSpec((1, 32), lambda i: (0, 0)),
            pl.BlockSpec((32, 16), lambda i: (0, 0)),
            pl.BlockSpec((1, 16), lambda i: (0, 0)),
            pl.BlockSpec((16, 1), lambda i: (0, 0)),
            pl.BlockSpec((1, 1), lambda i: (0, 0)),
        ],
        out_specs=pl.BlockSpec((MLP_BLOCK, 1), lambda i: (i, 0)),
        out_shape=jax.ShapeDtypeStruct((BATCH, 1), jnp.float32),
    )(u_rows, i_rows, u_par, i_par, W1a, W1b, b1, W2, b2, W3, b3)


def kernel(users, items, user_embedding, item_embedding, W1, b1, W2, b2, W3, b3):
    users = users.astype(jnp.int32)
    items = items.astype(jnp.int32)
    uidx2d = (users >> 1).reshape(IDX_ROWS, IDX_COLS)
    iidx2d = (items >> 1).reshape(IDX_ROWS, IDX_COLS)
    u_par = (users & 1).reshape(BATCH, 1)
    i_par = (items & 1).reshape(BATCH, 1)
    ue2 = user_embedding.reshape(-1, 2 * DIM)
    ie2 = item_embedding.reshape(-1, 2 * DIM)
    u_rows, i_rows = _sc_gather(uidx2d, iidx2d, ue2, ie2)
    u_rows = u_rows.reshape(BATCH, 2 * DIM)
    i_rows = i_rows.reshape(BATCH, 2 * DIM)
    out = _mlp(
        u_rows, i_rows, u_par, i_par,
        W1[:DIM], W1[DIM:],
        b1.reshape(1, 32),
        W2, b2.reshape(1, 16),
        W3, b3.reshape(1, 1),
    )
    return (out, out)


# speedup vs baseline: 1.5799x; 1.5799x over previous
import functools

import jax
import jax.numpy as jnp
from jax import lax
from jax.experimental import pallas as pl
from jax.experimental.pallas import tpu as pltpu
from jax.experimental.pallas import tpu_sc as plsc

BATCH = 16384
DIM = 64
NUM_WORKERS = 32
B_PER_W = BATCH // NUM_WORKERS

MLP_BLOCK = 2048


def _sc_gather_body(uidx_hbm, iidx_hbm, ue_hbm, ie_hbm, u_out, i_out,
                    uidx_v, iidx_v, rows_v, sem):
    nc = 2
    wid = lax.axis_index("s") * nc + lax.axis_index("c")
    base = wid * B_PER_W
    pltpu.sync_copy(uidx_hbm.at[pl.ds(base, B_PER_W)], uidx_v)
    pltpu.sync_copy(iidx_hbm.at[pl.ds(base, B_PER_W)], iidx_v)

    for tab, idx_v, out in ((ue_hbm, uidx_v, u_out), (ie_hbm, iidx_v, i_out)):
        def fire(g, carry, tab=tab, idx_v=idx_v):
            vec = idx_v[pl.ds(g * 16, 16)]
            for k in range(16):
                idx = lax.squeeze(lax.slice(vec, (k,), (k + 1,)), (0,))
                pltpu.async_copy(tab.at[pl.ds(idx, 1)],
                                 rows_v.at[pl.ds(g * 16 + k, 1)], sem)
            return carry

        lax.fori_loop(0, B_PER_W // 16, fire, 0)

        def drain(j, carry, tab=tab):
            pltpu.make_async_copy(tab.at[pl.ds(0, 1)],
                                  rows_v.at[pl.ds(0, 1)], sem).wait()
            return carry

        lax.fori_loop(0, B_PER_W, drain, 0)
        pltpu.sync_copy(rows_v, out.at[pl.ds(base, B_PER_W)])


def _sc_gather(users, items, user_embedding, item_embedding):
    mesh = plsc.VectorSubcoreMesh(core_axis_name="c", subcore_axis_name="s")
    kern = functools.partial(
        pl.kernel,
        mesh=mesh,
        out_type=[
            jax.ShapeDtypeStruct((BATCH, DIM), jnp.float32),
            jax.ShapeDtypeStruct((BATCH, DIM), jnp.float32),
        ],
        scratch_types=[
            pltpu.VMEM((B_PER_W,), jnp.int32),
            pltpu.VMEM((B_PER_W,), jnp.int32),
            pltpu.VMEM((B_PER_W, DIM), jnp.float32),
            pltpu.SemaphoreType.DMA,
        ],
    )(_sc_gather_body)
    return kern(users, items, user_embedding, item_embedding)


def _mlp_body(u_ref, i_ref, w1a_ref, w1b_ref, b1_ref, w2_ref, b2_ref,
              w3_ref, b3_ref, o_ref):
    h = jnp.dot(u_ref[...], w1a_ref[...], preferred_element_type=jnp.float32)
    h = h + jnp.dot(i_ref[...], w1b_ref[...], preferred_element_type=jnp.float32)
    h = jnp.maximum(h + b1_ref[...], 0.0)
    h = jnp.dot(h, w2_ref[...], preferred_element_type=jnp.float32) + b2_ref[...]
    h = jnp.maximum(h, 0.0)
    o = jnp.dot(h, w3_ref[...], preferred_element_type=jnp.float32) + b3_ref[...]
    o_ref[...] = 1.0 / (1.0 + jnp.exp(-o))


def _mlp(u_rows, i_rows, W1a, W1b, b1, W2, b2, W3, b3):
    n_blocks = BATCH // MLP_BLOCK
    return pl.pallas_call(
        _mlp_body,
        grid=(n_blocks,),
        in_specs=[
            pl.BlockSpec((MLP_BLOCK, DIM), lambda i: (i, 0)),
            pl.BlockSpec((MLP_BLOCK, DIM), lambda i: (i, 0)),
            pl.BlockSpec((DIM, 32), lambda i: (0, 0)),
            pl.BlockSpec((DIM, 32), lambda i: (0, 0)),
            pl.BlockSpec((1, 32), lambda i: (0, 0)),
            pl.BlockSpec((32, 16), lambda i: (0, 0)),
            pl.BlockSpec((1, 16), lambda i: (0, 0)),
            pl.BlockSpec((16, 1), lambda i: (0, 0)),
            pl.BlockSpec((1, 1), lambda i: (0, 0)),
        ],
        out_specs=pl.BlockSpec((MLP_BLOCK, 1), lambda i: (i, 0)),
        out_shape=jax.ShapeDtypeStruct((BATCH, 1), jnp.float32),
    )(u_rows, i_rows, W1a, W1b, b1, W2, b2, W3, b3)


def kernel(users, items, user_embedding, item_embedding, W1, b1, W2, b2, W3, b3):
    users = users.astype(jnp.int32)
    items = items.astype(jnp.int32)
    u_rows, i_rows = _sc_gather(users, items, user_embedding, item_embedding)
    out = _mlp(
        u_rows, i_rows,
        W1[:DIM], W1[DIM:],
        b1.reshape(1, 32),
        W2, b2.reshape(1, 16),
        W3, b3.reshape(1, 1),
    )
    return (out, out)


# speedup vs baseline: 1.6424x; 1.0396x over previous
import functools

import jax
import jax.numpy as jnp
from jax import lax
from jax.experimental import pallas as pl
from jax.experimental.pallas import tpu as pltpu
from jax.experimental.pallas import tpu_sc as plsc

BATCH = 16384
DIM = 64
NR_ROWS = 1000000
NUM_WORKERS = 32
B_PER_W = BATCH // NUM_WORKERS

L1_BLOCK = 2048
MLP_BLOCK = 2048


def _l1_body(uT_ref, iT_ref, w1a_ref, w1b_ref, o_ref):
    dn = (((0,), (0,)), ((), ()))
    a = lax.dot_general(uT_ref[...], w1a_ref[...], dn,
                        preferred_element_type=jnp.float32)
    b = lax.dot_general(iT_ref[...], w1b_ref[...], dn,
                        preferred_element_type=jnp.float32)
    o_ref[...] = jnp.concatenate([a, b], axis=1)


def _l1_table(uT, iT, W1a, W1b):
    n_blocks = (NR_ROWS + L1_BLOCK - 1) // L1_BLOCK
    return pl.pallas_call(
        _l1_body,
        grid=(n_blocks,),
        in_specs=[
            pl.BlockSpec((DIM, L1_BLOCK), lambda i: (0, i)),
            pl.BlockSpec((DIM, L1_BLOCK), lambda i: (0, i)),
            pl.BlockSpec((DIM, 32), lambda i: (0, 0)),
            pl.BlockSpec((DIM, 32), lambda i: (0, 0)),
        ],
        out_specs=pl.BlockSpec((L1_BLOCK, DIM), lambda i: (i, 0)),
        out_shape=jax.ShapeDtypeStruct((NR_ROWS, DIM), jnp.float32),
    )(uT, iT, W1a, W1b)


def _sc_gather_body(uidx_hbm, iidx_hbm, a_hbm, u_out, i_out,
                    uidx_v, iidx_v, rows_v, sem):
    nc = 2
    wid = lax.axis_index("s") * nc + lax.axis_index("c")
    base = wid * B_PER_W
    pltpu.sync_copy(uidx_hbm.at[pl.ds(base, B_PER_W)], uidx_v)
    pltpu.sync_copy(iidx_hbm.at[pl.ds(base, B_PER_W)], iidx_v)

    for idx_v, out in ((uidx_v, u_out), (iidx_v, i_out)):
        def fire(g, carry, idx_v=idx_v):
            vec = idx_v[pl.ds(g * 16, 16)]
            for k in range(16):
                idx = lax.squeeze(lax.slice(vec, (k,), (k + 1,)), (0,))
                pltpu.async_copy(a_hbm.at[pl.ds(idx, 1)],
                                 rows_v.at[pl.ds(g * 16 + k, 1)], sem)
            return carry

        lax.fori_loop(0, B_PER_W // 16, fire, 0)

        def drain(j, carry):
            pltpu.make_async_copy(a_hbm.at[pl.ds(0, 1)],
                                  rows_v.at[pl.ds(0, 1)], sem).wait()
            return carry

        lax.fori_loop(0, B_PER_W, drain, 0)
        pltpu.sync_copy(rows_v, out.at[pl.ds(base, B_PER_W)])


def _sc_gather(users, items, a_table):
    mesh = plsc.VectorSubcoreMesh(core_axis_name="c", subcore_axis_name="s")
    kern = functools.partial(
        pl.kernel,
        mesh=mesh,
        out_type=[
            jax.ShapeDtypeStruct((BATCH, DIM), jnp.float32),
            jax.ShapeDtypeStruct((BATCH, DIM), jnp.float32),
        ],
        scratch_types=[
            pltpu.VMEM((B_PER_W,), jnp.int32),
            pltpu.VMEM((B_PER_W,), jnp.int32),
            pltpu.VMEM((B_PER_W, DIM), jnp.float32),
            pltpu.SemaphoreType.DMA,
        ],
    )(_sc_gather_body)
    return kern(users, items, a_table)


def _mlp_body(u_ref, i_ref, b1_ref, w2_ref, b2_ref, w3_ref, b3_ref, o_ref):
    h = u_ref[:, :32] + i_ref[:, 32:] + b1_ref[...]
    h = jnp.maximum(h, 0.0)
    h = jnp.dot(h, w2_ref[...], preferred_element_type=jnp.float32) + b2_ref[...]
    h = jnp.maximum(h, 0.0)
    o = jnp.dot(h, w3_ref[...], preferred_element_type=jnp.float32) + b3_ref[...]
    o_ref[...] = 1.0 / (1.0 + jnp.exp(-o))


def _mlp(u_rows, i_rows, b1, W2, b2, W3, b3):
    n_blocks = BATCH // MLP_BLOCK
    return pl.pallas_call(
        _mlp_body,
        grid=(n_blocks,),
        in_specs=[
            pl.BlockSpec((MLP_BLOCK, DIM), lambda i: (i, 0)),
            pl.BlockSpec((MLP_BLOCK, DIM), lambda i: (i, 0)),
            pl.BlockSpec((1, 32), lambda i: (0, 0)),
            pl.BlockSpec((32, 16), lambda i: (0, 0)),
            pl.BlockSpec((1, 16), lambda i: (0, 0)),
            pl.BlockSpec((16, 1), lambda i: (0, 0)),
            pl.BlockSpec((1, 1), lambda i: (0, 0)),
        ],
        out_specs=pl.BlockSpec((MLP_BLOCK, 1), lambda i: (i, 0)),
        out_shape=jax.ShapeDtypeStruct((BATCH, 1), jnp.float32),
    )(u_rows, i_rows, b1, W2, b2, W3, b3)


def kernel(users, items, user_embedding, item_embedding, W1, b1, W2, b2, W3, b3):
    users = users.astype(jnp.int32)
    items = items.astype(jnp.int32)
    a_table = _l1_table(user_embedding.T, item_embedding.T, W1[:DIM], W1[DIM:])
    u_rows, i_rows = _sc_gather(users, items, a_table)
    out = _mlp(
        u_rows, i_rows,
        b1.reshape(1, 32),
        W2, b2.reshape(1, 16),
        W3, b3.reshape(1, 1),
    )
    return (out, out)


# speedup vs baseline: 2.4807x; 1.5104x over previous
import functools

import jax
import jax.numpy as jnp
from jax import lax
from jax.experimental import pallas as pl
from jax.experimental.pallas import tpu as pltpu
from jax.experimental.pallas import tpu_sc as plsc

BATCH = 16384
DIM = 64
NR_ROWS = 1000000
NUM_WORKERS = 32
B_PER_W = BATCH // NUM_WORKERS

L1_BLOCK = 4096
MLP_BLOCK = 2048


def _l1_body(uT_ref, iT_ref, w1aT_ref, w1bT_ref, o_ref):
    a = jnp.dot(w1aT_ref[...], uT_ref[...], preferred_element_type=jnp.float32)
    b = jnp.dot(w1bT_ref[...], iT_ref[...], preferred_element_type=jnp.float32)
    o_ref[...] = lax.transpose(jnp.concatenate([a, b], axis=0), (1, 0))


def _l1_table(uT, iT, W1aT, W1bT):
    n_blocks = (NR_ROWS + L1_BLOCK - 1) // L1_BLOCK
    return pl.pallas_call(
        _l1_body,
        grid=(n_blocks,),
        in_specs=[
            pl.BlockSpec((DIM, L1_BLOCK), lambda i: (0, i)),
            pl.BlockSpec((DIM, L1_BLOCK), lambda i: (0, i)),
            pl.BlockSpec((32, DIM), lambda i: (0, 0)),
            pl.BlockSpec((32, DIM), lambda i: (0, 0)),
        ],
        out_specs=pl.BlockSpec((L1_BLOCK, DIM), lambda i: (i, 0)),
        out_shape=jax.ShapeDtypeStruct((NR_ROWS, DIM), jnp.float32),
    )(uT, iT, W1aT, W1bT)


def _sc_gather_body(uidx_hbm, iidx_hbm, a_hbm, u_out, i_out,
                    uidx_v, iidx_v, rows_v, sem):
    nc = 2
    wid = lax.axis_index("s") * nc + lax.axis_index("c")
    base = wid * B_PER_W
    pltpu.sync_copy(uidx_hbm.at[pl.ds(base, B_PER_W)], uidx_v)
    pltpu.sync_copy(iidx_hbm.at[pl.ds(base, B_PER_W)], iidx_v)

    for idx_v, out in ((uidx_v, u_out), (iidx_v, i_out)):
        def fire(g, carry, idx_v=idx_v):
            vec = idx_v[pl.ds(g * 16, 16)]
            for k in range(16):
                idx = lax.squeeze(lax.slice(vec, (k,), (k + 1,)), (0,))
                pltpu.async_copy(a_hbm.at[pl.ds(idx, 1)],
                                 rows_v.at[pl.ds(g * 16 + k, 1)], sem)
            return carry

        lax.fori_loop(0, B_PER_W // 16, fire, 0)

        def drain(j, carry):
            pltpu.make_async_copy(a_hbm.at[pl.ds(0, 1)],
                                  rows_v.at[pl.ds(0, 1)], sem).wait()
            return carry

        lax.fori_loop(0, B_PER_W, drain, 0)
        pltpu.sync_copy(rows_v, out.at[pl.ds(base, B_PER_W)])


def _sc_gather(users, items, a_table):
    mesh = plsc.VectorSubcoreMesh(core_axis_name="c", subcore_axis_name="s")
    kern = functools.partial(
        pl.kernel,
        mesh=mesh,
        out_type=[
            jax.ShapeDtypeStruct((BATCH, DIM), jnp.float32),
            jax.ShapeDtypeStruct((BATCH, DIM), jnp.float32),
        ],
        scratch_types=[
            pltpu.VMEM((B_PER_W,), jnp.int32),
            pltpu.VMEM((B_PER_W,), jnp.int32),
            pltpu.VMEM((B_PER_W, DIM), jnp.float32),
            pltpu.SemaphoreType.DMA,
        ],
    )(_sc_gather_body)
    return kern(users, items, a_table)


def _mlp_body(u_ref, i_ref, b1_ref, w2_ref, b2_ref, w3_ref, b3_ref, o_ref):
    h = u_ref[:, :32] + i_ref[:, 32:] + b1_ref[...]
    h = jnp.maximum(h, 0.0)
    h = jnp.dot(h, w2_ref[...], preferred_element_type=jnp.float32) + b2_ref[...]
    h = jnp.maximum(h, 0.0)
    o = jnp.dot(h, w3_ref[...], preferred_element_type=jnp.float32) + b3_ref[...]
    o_ref[...] = 1.0 / (1.0 + jnp.exp(-o))


def _mlp(u_rows, i_rows, b1, W2, b2, W3, b3):
    n_blocks = BATCH // MLP_BLOCK
    return pl.pallas_call(
        _mlp_body,
        grid=(n_blocks,),
        in_specs=[
            pl.BlockSpec((MLP_BLOCK, DIM), lambda i: (i, 0)),
            pl.BlockSpec((MLP_BLOCK, DIM), lambda i: (i, 0)),
            pl.BlockSpec((1, 32), lambda i: (0, 0)),
            pl.BlockSpec((32, 16), lambda i: (0, 0)),
            pl.BlockSpec((1, 16), lambda i: (0, 0)),
            pl.BlockSpec((16, 1), lambda i: (0, 0)),
            pl.BlockSpec((1, 1), lambda i: (0, 0)),
        ],
        out_specs=pl.BlockSpec((MLP_BLOCK, 1), lambda i: (i, 0)),
        out_shape=jax.ShapeDtypeStruct((BATCH, 1), jnp.float32),
    )(u_rows, i_rows, b1, W2, b2, W3, b3)


def kernel(users, items, user_embedding, item_embedding, W1, b1, W2, b2, W3, b3):
    users = users.astype(jnp.int32)
    items = items.astype(jnp.int32)
    W1T = W1.T
    a_table = _l1_table(user_embedding.T, item_embedding.T,
                        W1T[:, :DIM], W1T[:, DIM:])
    u_rows, i_rows = _sc_gather(users, items, a_table)
    out = _mlp(
        u_rows, i_rows,
        b1.reshape(1, 32),
        W2, b2.reshape(1, 16),
        W3, b3.reshape(1, 1),
    )
    return (out, out)


# speedup vs baseline: 2.9109x; 1.1734x over previous
import functools

import jax
import jax.numpy as jnp
from jax import lax
from jax.experimental import pallas as pl
from jax.experimental.pallas import tpu as pltpu
from jax.experimental.pallas import tpu_sc as plsc

BATCH = 16384
DIM = 64
NR_ROWS = 1000000
SPLIT = 1 << 19
NUM_WORKERS = 32
B_PER_W = BATCH // NUM_WORKERS

L1_BLOCK = 4096
MLP_BLOCK = 2048


def _l1_body(uTt_ref, uTb_ref, iTt_ref, iTb_ref, w1aT_ref, w1bT_ref, o_ref):
    at = jnp.dot(w1aT_ref[...], uTt_ref[...], preferred_element_type=jnp.float32)
    bt = jnp.dot(w1bT_ref[...], iTt_ref[...], preferred_element_type=jnp.float32)
    ab = jnp.dot(w1aT_ref[...], uTb_ref[...], preferred_element_type=jnp.float32)
    bb = jnp.dot(w1bT_ref[...], iTb_ref[...], preferred_element_type=jnp.float32)
    zt = lax.transpose(jnp.concatenate([at, bt], axis=0), (1, 0))
    zb = lax.transpose(jnp.concatenate([ab, bb], axis=0), (1, 0))
    o_ref[...] = jnp.concatenate([zt, zb], axis=1)


def _l1_table(uT, iT, W1aT, W1bT):
    n_blocks = SPLIT // L1_BLOCK
    shift = SPLIT // L1_BLOCK
    last = NR_ROWS // L1_BLOCK
    clamped = lambda i: (0, jnp.minimum(i + shift, last))
    return pl.pallas_call(
        _l1_body,
        grid=(n_blocks,),
        in_specs=[
            pl.BlockSpec((DIM, L1_BLOCK), lambda i: (0, i)),
            pl.BlockSpec((DIM, L1_BLOCK), clamped),
            pl.BlockSpec((DIM, L1_BLOCK), lambda i: (0, i)),
            pl.BlockSpec((DIM, L1_BLOCK), clamped),
            pl.BlockSpec((32, DIM), lambda i: (0, 0)),
            pl.BlockSpec((32, DIM), lambda i: (0, 0)),
        ],
        out_specs=pl.BlockSpec((L1_BLOCK, 2 * DIM), lambda i: (i, 0)),
        out_shape=jax.ShapeDtypeStruct((SPLIT, 2 * DIM), jnp.float32),
    )(uT, uT, iT, iT, W1aT, W1bT)


def _sc_gather_body(uidx_hbm, iidx_hbm, a_hbm, u_out, i_out,
                    uidx_v, iidx_v, rows_v, sem):
    nc = 2
    wid = lax.axis_index("s") * nc + lax.axis_index("c")
    base = wid * B_PER_W
    pltpu.sync_copy(uidx_hbm.at[pl.ds(base, B_PER_W)], uidx_v)
    pltpu.sync_copy(iidx_hbm.at[pl.ds(base, B_PER_W)], iidx_v)

    for idx_v, out in ((uidx_v, u_out), (iidx_v, i_out)):
        def fire(g, carry, idx_v=idx_v):
            vec = idx_v[pl.ds(g * 16, 16)]
            for k in range(16):
                idx = lax.squeeze(lax.slice(vec, (k,), (k + 1,)), (0,))
                pltpu.async_copy(a_hbm.at[pl.ds(idx, 1)],
                                 rows_v.at[pl.ds(g * 16 + k, 1)], sem)
            return carry

        lax.fori_loop(0, B_PER_W // 16, fire, 0)

        def drain(j, carry):
            pltpu.make_async_copy(a_hbm.at[pl.ds(0, 1)],
                                  rows_v.at[pl.ds(0, 1)], sem).wait()
            return carry

        lax.fori_loop(0, B_PER_W, drain, 0)
        pltpu.sync_copy(rows_v, out.at[pl.ds(base, B_PER_W)])


def _sc_gather(uidx, iidx, a_table):
    mesh = plsc.VectorSubcoreMesh(core_axis_name="c", subcore_axis_name="s")
    kern = functools.partial(
        pl.kernel,
        mesh=mesh,
        out_type=[
            jax.ShapeDtypeStruct((BATCH, 2 * DIM), jnp.float32),
            jax.ShapeDtypeStruct((BATCH, 2 * DIM), jnp.float32),
        ],
        scratch_types=[
            pltpu.VMEM((B_PER_W,), jnp.int32),
            pltpu.VMEM((B_PER_W,), jnp.int32),
            pltpu.VMEM((B_PER_W, 2 * DIM), jnp.float32),
            pltpu.SemaphoreType.DMA,
        ],
    )(_sc_gather_body)
    return kern(uidx, iidx, a_table)


def _mlp_body(u_ref, i_ref, fu_ref, fi_ref, b1_ref, w2_ref, b2_ref,
              w3_ref, b3_ref, o_ref):
    zu = jnp.where(fu_ref[...] == 1, u_ref[:, 64:96], u_ref[:, 0:32])
    zi = jnp.where(fi_ref[...] == 1, i_ref[:, 96:128], i_ref[:, 32:64])
    h = zu + zi + b1_ref[...]
    h = jnp.maximum(h, 0.0)
    h = jnp.dot(h, w2_ref[...], preferred_element_type=jnp.float32) + b2_ref[...]
    h = jnp.maximum(h, 0.0)
    o = jnp.dot(h, w3_ref[...], preferred_element_type=jnp.float32) + b3_ref[...]
    o_ref[...] = 1.0 / (1.0 + jnp.exp(-o))


def _mlp(u_rows, i_rows, fu, fi, b1, W2, b2, W3, b3):
    n_blocks = BATCH // MLP_BLOCK
    return pl.pallas_call(
        _mlp_body,
        grid=(n_blocks,),
        in_specs=[
            pl.BlockSpec((MLP_BLOCK, 2 * DIM), lambda i: (i, 0)),
            pl.BlockSpec((MLP_BLOCK, 2 * DIM), lambda i: (i, 0)),
            pl.BlockSpec((MLP_BLOCK, 1), lambda i: (i, 0)),
            pl.BlockSpec((MLP_BLOCK, 1), lambda i: (i, 0)),
            pl.BlockSpec((1, 32), lambda i: (0, 0)),
            pl.BlockSpec((32, 16), lambda i: (0, 0)),
            pl.BlockSpec((1, 16), lambda i: (0, 0)),
            pl.BlockSpec((16, 1), lambda i: (0, 0)),
            pl.BlockSpec((1, 1), lambda i: (0, 0)),
        ],
        out_specs=pl.BlockSpec((MLP_BLOCK, 1), lambda i: (i, 0)),
        out_shape=jax.ShapeDtypeStruct((BATCH, 1), jnp.float32),
    )(u_rows, i_rows, fu, fi, b1, W2, b2, W3, b3)


def kernel(users, items, user_embedding, item_embedding, W1, b1, W2, b2, W3, b3):
    users = users.astype(jnp.int32)
    items = items.astype(jnp.int32)
    W1T = W1.T
    a_table = _l1_table(user_embedding.T, item_embedding.T,
                        W1T[:, :DIM], W1T[:, DIM:])
    uidx = users & (SPLIT - 1)
    iidx = items & (SPLIT - 1)
    fu = (users >> 19).reshape(BATCH, 1)
    fi = (items >> 19).reshape(BATCH, 1)
    u_rows, i_rows = _sc_gather(uidx, iidx, a_table)
    out = _mlp(
        u_rows, i_rows, fu, fi,
        b1.reshape(1, 32),
        W2, b2.reshape(1, 16),
        W3, b3.reshape(1, 1),
    )
    return (out, out)


# speedup vs baseline: 3.2928x; 1.1312x over previous
import functools

import jax
import jax.numpy as jnp
from jax import lax
from jax.experimental import pallas as pl
from jax.experimental.pallas import tpu as pltpu
from jax.experimental.pallas import tpu_sc as plsc

BATCH = 16384
DIM = 64
NR_ROWS = 1000000
SPLIT = 1 << 19
NUM_WORKERS = 32
B_PER_W = BATCH // NUM_WORKERS

L1_BLOCK = 8192
MLP_BLOCK = 2048


def _l1_body(uTt_ref, uTb_ref, iTt_ref, iTb_ref, w1aT_ref, w1bT_ref, o_ref):
    at = jnp.dot(w1aT_ref[...], uTt_ref[...], preferred_element_type=jnp.float32)
    bt = jnp.dot(w1bT_ref[...], iTt_ref[...], preferred_element_type=jnp.float32)
    ab = jnp.dot(w1aT_ref[...], uTb_ref[...], preferred_element_type=jnp.float32)
    bb = jnp.dot(w1bT_ref[...], iTb_ref[...], preferred_element_type=jnp.float32)
    zt = lax.transpose(jnp.concatenate([at, bt], axis=0), (1, 0))
    zb = lax.transpose(jnp.concatenate([ab, bb], axis=0), (1, 0))
    o_ref[...] = jnp.concatenate([zt, zb], axis=1)


def _l1_table(uT, iT, W1aT, W1bT):
    n_blocks = SPLIT // L1_BLOCK
    shift = SPLIT // L1_BLOCK
    last = NR_ROWS // L1_BLOCK
    clamped = lambda i: (0, jnp.minimum(i + shift, last))
    return pl.pallas_call(
        _l1_body,
        grid=(n_blocks,),
        in_specs=[
            pl.BlockSpec((DIM, L1_BLOCK), lambda i: (0, i)),
            pl.BlockSpec((DIM, L1_BLOCK), clamped),
            pl.BlockSpec((DIM, L1_BLOCK), lambda i: (0, i)),
            pl.BlockSpec((DIM, L1_BLOCK), clamped),
            pl.BlockSpec((32, DIM), lambda i: (0, 0)),
            pl.BlockSpec((32, DIM), lambda i: (0, 0)),
        ],
        out_specs=pl.BlockSpec((L1_BLOCK, 2 * DIM), lambda i: (i, 0)),
        out_shape=jax.ShapeDtypeStruct((SPLIT, 2 * DIM), jnp.float32),
    )(uT, uT, iT, iT, W1aT, W1bT)


def _sc_gather_body(uidx_hbm, iidx_hbm, a_hbm, u_out, i_out,
                    uidx_v, iidx_v, rows_v, sem):
    nc = 2
    wid = lax.axis_index("s") * nc + lax.axis_index("c")
    base = wid * B_PER_W
    pltpu.sync_copy(uidx_hbm.at[pl.ds(base, B_PER_W)], uidx_v)
    pltpu.sync_copy(iidx_hbm.at[pl.ds(base, B_PER_W)], iidx_v)

    for idx_v, out in ((uidx_v, u_out), (iidx_v, i_out)):
        def fire(g, carry, idx_v=idx_v):
            vec = idx_v[pl.ds(g * 16, 16)]
            for k in range(16):
                idx = lax.squeeze(lax.slice(vec, (k,), (k + 1,)), (0,))
                pltpu.async_copy(a_hbm.at[pl.ds(idx, 1)],
                                 rows_v.at[pl.ds(g * 16 + k, 1)], sem)
            return carry

        lax.fori_loop(0, B_PER_W // 16, fire, 0)

        def drain(j, carry):
            pltpu.make_async_copy(a_hbm.at[pl.ds(0, 1)],
                                  rows_v.at[pl.ds(0, 1)], sem).wait()
            return carry

        lax.fori_loop(0, B_PER_W, drain, 0)
        pltpu.sync_copy(rows_v, out.at[pl.ds(base, B_PER_W)])


def _sc_gather(uidx, iidx, a_table):
    mesh = plsc.VectorSubcoreMesh(core_axis_name="c", subcore_axis_name="s")
    kern = functools.partial(
        pl.kernel,
        mesh=mesh,
        out_type=[
            jax.ShapeDtypeStruct((BATCH, 2 * DIM), jnp.float32),
            jax.ShapeDtypeStruct((BATCH, 2 * DIM), jnp.float32),
        ],
        scratch_types=[
            pltpu.VMEM((B_PER_W,), jnp.int32),
            pltpu.VMEM((B_PER_W,), jnp.int32),
            pltpu.VMEM((B_PER_W, 2 * DIM), jnp.float32),
            pltpu.SemaphoreType.DMA,
        ],
    )(_sc_gather_body)
    return kern(uidx, iidx, a_table)


def _mlp_body(u_ref, i_ref, fu_ref, fi_ref, b1_ref, w2_ref, b2_ref,
              w3_ref, b3_ref, o_ref):
    zu = jnp.where(fu_ref[...] == 1, u_ref[:, 64:96], u_ref[:, 0:32])
    zi = jnp.where(fi_ref[...] == 1, i_ref[:, 96:128], i_ref[:, 32:64])
    h = zu + zi + b1_ref[...]
    h = jnp.maximum(h, 0.0)
    h = jnp.dot(h, w2_ref[...], preferred_element_type=jnp.float32) + b2_ref[...]
    h = jnp.maximum(h, 0.0)
    o = jnp.dot(h, w3_ref[...], preferred_element_type=jnp.float32) + b3_ref[...]
    o_ref[...] = 1.0 / (1.0 + jnp.exp(-o))


def _mlp(u_rows, i_rows, fu, fi, b1, W2, b2, W3, b3):
    n_blocks = BATCH // MLP_BLOCK
    return pl.pallas_call(
        _mlp_body,
        grid=(n_blocks,),
        in_specs=[
            pl.BlockSpec((MLP_BLOCK, 2 * DIM), lambda i: (i, 0)),
            pl.BlockSpec((MLP_BLOCK, 2 * DIM), lambda i: (i, 0)),
            pl.BlockSpec((MLP_BLOCK, 1), lambda i: (i, 0)),
            pl.BlockSpec((MLP_BLOCK, 1), lambda i: (i, 0)),
            pl.BlockSpec((1, 32), lambda i: (0, 0)),
            pl.BlockSpec((32, 16), lambda i: (0, 0)),
            pl.BlockSpec((1, 16), lambda i: (0, 0)),
            pl.BlockSpec((16, 1), lambda i: (0, 0)),
            pl.BlockSpec((1, 1), lambda i: (0, 0)),
        ],
        out_specs=pl.BlockSpec((MLP_BLOCK, 1), lambda i: (i, 0)),
        out_shape=jax.ShapeDtypeStruct((BATCH, 1), jnp.float32),
    )(u_rows, i_rows, fu, fi, b1, W2, b2, W3, b3)


def kernel(users, items, user_embedding, item_embedding, W1, b1, W2, b2, W3, b3):
    users = users.astype(jnp.int32)
    items = items.astype(jnp.int32)
    W1T = W1.T
    a_table = _l1_table(user_embedding.T, item_embedding.T,
                        W1T[:, :DIM], W1T[:, DIM:])
    uidx = users & (SPLIT - 1)
    iidx = items & (SPLIT - 1)
    fu = (users >> 19).reshape(BATCH, 1)
    fi = (items >> 19).reshape(BATCH, 1)
    u_rows, i_rows = _sc_gather(uidx, iidx, a_table)
    out = _mlp(
        u_rows, i_rows, fu, fi,
        b1.reshape(1, 32),
        W2, b2.reshape(1, 16),
        W3, b3.reshape(1, 1),
    )
    return (out, out)


# speedup vs baseline: 3.4396x; 1.0446x over previous
import functools

import jax
import jax.numpy as jnp
from jax import lax
from jax.experimental import pallas as pl
from jax.experimental.pallas import tpu as pltpu
from jax.experimental.pallas import tpu_sc as plsc

BATCH = 16384
DIM = 64
NR_ROWS = 1000000
SPLIT = 1 << 19
NUM_WORKERS = 32
B_PER_W = BATCH // NUM_WORKERS

L1_BLOCK = 16384
MLP_BLOCK = 2048


def _l1_body(uTt_ref, uTb_ref, iTt_ref, iTb_ref, w1aT_ref, w1bT_ref, o_ref):
    at = jnp.dot(w1aT_ref[...], uTt_ref[...], preferred_element_type=jnp.float32)
    bt = jnp.dot(w1bT_ref[...], iTt_ref[...], preferred_element_type=jnp.float32)
    ab = jnp.dot(w1aT_ref[...], uTb_ref[...], preferred_element_type=jnp.float32)
    bb = jnp.dot(w1bT_ref[...], iTb_ref[...], preferred_element_type=jnp.float32)
    zt = lax.transpose(jnp.concatenate([at, bt], axis=0), (1, 0))
    zb = lax.transpose(jnp.concatenate([ab, bb], axis=0), (1, 0))
    o_ref[...] = jnp.concatenate([zt, zb], axis=1)


def _l1_table(uT, iT, W1aT, W1bT):
    n_blocks = SPLIT // L1_BLOCK
    shift = SPLIT // L1_BLOCK
    last = NR_ROWS // L1_BLOCK
    clamped = lambda i: (0, jnp.minimum(i + shift, last))
    return pl.pallas_call(
        _l1_body,
        grid=(n_blocks,),
        in_specs=[
            pl.BlockSpec((DIM, L1_BLOCK), lambda i: (0, i)),
            pl.BlockSpec((DIM, L1_BLOCK), clamped),
            pl.BlockSpec((DIM, L1_BLOCK), lambda i: (0, i)),
            pl.BlockSpec((DIM, L1_BLOCK), clamped),
            pl.BlockSpec((32, DIM), lambda i: (0, 0)),
            pl.BlockSpec((32, DIM), lambda i: (0, 0)),
        ],
        out_specs=pl.BlockSpec((L1_BLOCK, 2 * DIM), lambda i: (i, 0)),
        out_shape=jax.ShapeDtypeStruct((SPLIT, 2 * DIM), jnp.float32),
        compiler_params=pltpu.CompilerParams(
            vmem_limit_bytes=120 * 1024 * 1024),
    )(uT, uT, iT, iT, W1aT, W1bT)


def _sc_gather_body(uidx_hbm, iidx_hbm, a_hbm, u_out, i_out,
                    uidx_v, iidx_v, rows_v, sem):
    nc = 2
    wid = lax.axis_index("s") * nc + lax.axis_index("c")
    base = wid * B_PER_W
    pltpu.sync_copy(uidx_hbm.at[pl.ds(base, B_PER_W)], uidx_v)
    pltpu.sync_copy(iidx_hbm.at[pl.ds(base, B_PER_W)], iidx_v)

    for idx_v, out in ((uidx_v, u_out), (iidx_v, i_out)):
        def fire(g, carry, idx_v=idx_v):
            vec = idx_v[pl.ds(g * 16, 16)]
            for k in range(16):
                idx = lax.squeeze(lax.slice(vec, (k,), (k + 1,)), (0,))
                pltpu.async_copy(a_hbm.at[pl.ds(idx, 1)],
                                 rows_v.at[pl.ds(g * 16 + k, 1)], sem)
            return carry

        lax.fori_loop(0, B_PER_W // 16, fire, 0)

        def drain(j, carry):
            pltpu.make_async_copy(a_hbm.at[pl.ds(0, 1)],
                                  rows_v.at[pl.ds(0, 1)], sem).wait()
            return carry

        lax.fori_loop(0, B_PER_W, drain, 0)
        pltpu.sync_copy(rows_v, out.at[pl.ds(base, B_PER_W)])


def _sc_gather(uidx, iidx, a_table):
    mesh = plsc.VectorSubcoreMesh(core_axis_name="c", subcore_axis_name="s")
    kern = functools.partial(
        pl.kernel,
        mesh=mesh,
        out_type=[
            jax.ShapeDtypeStruct((BATCH, 2 * DIM), jnp.float32),
            jax.ShapeDtypeStruct((BATCH, 2 * DIM), jnp.float32),
        ],
        scratch_types=[
            pltpu.VMEM((B_PER_W,), jnp.int32),
            pltpu.VMEM((B_PER_W,), jnp.int32),
            pltpu.VMEM((B_PER_W, 2 * DIM), jnp.float32),
            pltpu.SemaphoreType.DMA,
        ],
    )(_sc_gather_body)
    return kern(uidx, iidx, a_table)


def _mlp_body(u_ref, i_ref, fu_ref, fi_ref, b1_ref, w2_ref, b2_ref,
              w3_ref, b3_ref, o_ref):
    zu = jnp.where(fu_ref[...] == 1, u_ref[:, 64:96], u_ref[:, 0:32])
    zi = jnp.where(fi_ref[...] == 1, i_ref[:, 96:128], i_ref[:, 32:64])
    h = zu + zi + b1_ref[...]
    h = jnp.maximum(h, 0.0)
    h = jnp.dot(h, w2_ref[...], preferred_element_type=jnp.float32) + b2_ref[...]
    h = jnp.maximum(h, 0.0)
    o = jnp.dot(h, w3_ref[...], preferred_element_type=jnp.float32) + b3_ref[...]
    o_ref[...] = 1.0 / (1.0 + jnp.exp(-o))


def _mlp(u_rows, i_rows, fu, fi, b1, W2, b2, W3, b3):
    n_blocks = BATCH // MLP_BLOCK
    return pl.pallas_call(
        _mlp_body,
        grid=(n_blocks,),
        in_specs=[
            pl.BlockSpec((MLP_BLOCK, 2 * DIM), lambda i: (i, 0)),
            pl.BlockSpec((MLP_BLOCK, 2 * DIM), lambda i: (i, 0)),
            pl.BlockSpec((MLP_BLOCK, 1), lambda i: (i, 0)),
            pl.BlockSpec((MLP_BLOCK, 1), lambda i: (i, 0)),
            pl.BlockSpec((1, 32), lambda i: (0, 0)),
            pl.BlockSpec((32, 16), lambda i: (0, 0)),
            pl.BlockSpec((1, 16), lambda i: (0, 0)),
            pl.BlockSpec((16, 1), lambda i: (0, 0)),
            pl.BlockSpec((1, 1), lambda i: (0, 0)),
        ],
        out_specs=pl.BlockSpec((MLP_BLOCK, 1), lambda i: (i, 0)),
        out_shape=jax.ShapeDtypeStruct((BATCH, 1), jnp.float32),
    )(u_rows, i_rows, fu, fi, b1, W2, b2, W3, b3)


def kernel(users, items, user_embedding, item_embedding, W1, b1, W2, b2, W3, b3):
    users = users.astype(jnp.int32)
    items = items.astype(jnp.int32)
    W1T = W1.T
    a_table = _l1_table(user_embedding.T, item_embedding.T,
                        W1T[:, :DIM], W1T[:, DIM:])
    uidx = users & (SPLIT - 1)
    iidx = items & (SPLIT - 1)
    fu = (users >> 19).reshape(BATCH, 1)
    fi = (items >> 19).reshape(BATCH, 1)
    u_rows, i_rows = _sc_gather(uidx, iidx, a_table)
    out = _mlp(
        u_rows, i_rows, fu, fi,
        b1.reshape(1, 32),
        W2, b2.reshape(1, 16),
        W3, b3.reshape(1, 1),
    )
    return (out, out)
